# Initial kernel scaffold; baseline (speedup 1.0000x reference)
import jax
import jax.numpy as jnp
from jax.experimental import pallas as pl


def kernel(blocks, targets, table):
    raise NotImplementedError("write your pallas kernel here")


# speedup vs baseline: 2.0114x; 2.0114x over previous
import functools

import jax
import jax.numpy as jnp
from jax import lax
from jax.experimental import pallas as pl
from jax.experimental.pallas import tpu as pltpu
from jax.experimental.pallas import tpu_sc as plsc

_V = 8192
_N = 8192
_ROWS_BLK = 128
_NC = 2
_NS = 16
_NW = _NC * _NS
_CHUNK = _N // _NW
_L = 16


def _lse_body(tbl_ref, out_ref):
    x = tbl_ref[...]
    m = jnp.max(x, axis=1)
    s = jnp.sum(jnp.exp(x - m[:, None]), axis=1)
    i = pl.program_id(0)
    out_ref[pl.ds(i, 1), :] = (m + jnp.log(s)).reshape(1, _ROWS_BLK)


def _row_lse(table):
    grid = _V // _ROWS_BLK
    out = pl.pallas_call(
        _lse_body,
        grid=(grid,),
        in_specs=[pl.BlockSpec((_ROWS_BLK, _V), lambda i: (i, 0))],
        out_specs=pl.BlockSpec((grid, _ROWS_BLK), lambda i: (0, 0)),
        out_shape=jax.ShapeDtypeStruct((grid, _ROWS_BLK), jnp.float32),
    )(table)
    return out.reshape(-1)


@functools.cache
def _make_sc_gather():
    mesh = plsc.VectorSubcoreMesh(core_axis_name="c", subcore_axis_name="s")
    return functools.partial(
        pl.kernel,
        mesh=mesh,
        out_type=jax.ShapeDtypeStruct((_NW, _L), jnp.float32),
        scratch_types=[
            pltpu.VMEM((_CHUNK // 128, 128), jnp.int32),
            pltpu.VMEM((_CHUNK // 128, 128), jnp.int32),
            pltpu.VMEM((_CHUNK // 128, 128), jnp.int32),
            pltpu.VMEM((_CHUNK // 128, 128), jnp.float32),
            pltpu.VMEM((_CHUNK // 128, 128), jnp.float32),
            pltpu.VMEM((_L,), jnp.float32),
            pltpu.SemaphoreType.DMA,
        ],
    )(_sc_gather_body)


def _sc_gather_body(blocks_hbm, targets_hbm, tbl_hbm, lse_hbm, out_hbm,
                    b_v, t_v, idx_v, val_v, lseval_v, acc_v, sem):
    wid = lax.axis_index("s") * _NC + lax.axis_index("c")
    base = wid * _CHUNK
    n_rows = _CHUNK // 128
    for j in range(n_rows):
        pltpu.sync_copy(blocks_hbm.at[pl.ds(base + j * 128, 128)], b_v.at[j])
        pltpu.sync_copy(targets_hbm.at[pl.ds(base + j * 128, 128)], t_v.at[j])

    for j in range(n_rows):
        for i in range(128 // _L):
            bb = b_v[j, pl.ds(i * _L, _L)]
            tt = t_v[j, pl.ds(i * _L, _L)]
            idx_v[j, pl.ds(i * _L, _L)] = bb * _V + tt
    copies = []
    for j in range(n_rows):
        copies.append(pltpu.async_copy(tbl_hbm.at[idx_v.at[j]], val_v.at[j], sem))
        copies.append(pltpu.async_copy(lse_hbm.at[b_v.at[j]], lseval_v.at[j], sem))
    for c in copies:
        c.wait()

    acc = jnp.zeros((_L,), jnp.float32)
    for j in range(n_rows):
        for i in range(128 // _L):
            sl = pl.ds(i * _L, _L)
            acc = acc + (lseval_v[j, sl] - val_v[j, sl])
    acc_v[...] = acc
    pltpu.sync_copy(acc_v, out_hbm.at[wid])


def kernel(blocks, targets, table):
    blocks_f = blocks.reshape(-1).astype(jnp.int32)
    targets_f = targets.reshape(-1).astype(jnp.int32)
    lse = _row_lse(table)
    parts = _make_sc_gather()(blocks_f, targets_f, table.reshape(-1), lse)
    return jnp.sum(parts) / jnp.float32(_N)


# speedup vs baseline: 2.8117x; 1.3979x over previous
import functools

import jax
import jax.numpy as jnp
from jax import lax
from jax.experimental import pallas as pl
from jax.experimental.pallas import tpu as pltpu
from jax.experimental.pallas import tpu_sc as plsc

_V = 8192
_N = 8192
_ROWS_BLK = 128
_NC = 2
_NS = 16
_NW = _NC * _NS
_CHUNK = _N // _NW
_L = 16


def _lse_body(tbl_ref, out_ref, flat_ref):
    x = tbl_ref[...]
    m = jnp.max(x, axis=1)
    s = jnp.sum(jnp.exp(x - m[:, None]), axis=1)
    i = pl.program_id(0)
    out_ref[pl.ds(i, 1), :] = (m + jnp.log(s)).reshape(1, _ROWS_BLK)
    flat_ref[...] = x.reshape(-1)


def _row_lse(table):
    grid = _V // _ROWS_BLK
    out, flat = pl.pallas_call(
        _lse_body,
        grid=(grid,),
        in_specs=[pl.BlockSpec((_ROWS_BLK, _V), lambda i: (i, 0))],
        out_specs=[
            pl.BlockSpec((grid, _ROWS_BLK), lambda i: (0, 0)),
            pl.BlockSpec((_ROWS_BLK * _V,), lambda i: (i,)),
        ],
        out_shape=[
            jax.ShapeDtypeStruct((grid, _ROWS_BLK), jnp.float32),
            jax.ShapeDtypeStruct((_V * _V,), jnp.float32),
        ],
    )(table)
    return out.reshape(-1), flat


@functools.cache
def _make_sc_gather():
    mesh = plsc.VectorSubcoreMesh(core_axis_name="c", subcore_axis_name="s")
    return functools.partial(
        pl.kernel,
        mesh=mesh,
        out_type=jax.ShapeDtypeStruct((_NW, _L), jnp.float32),
        scratch_types=[
            pltpu.VMEM((_CHUNK // 128, 128), jnp.int32),
            pltpu.VMEM((_CHUNK // 128, 128), jnp.int32),
            pltpu.VMEM((_CHUNK // 128, 128), jnp.int32),
            pltpu.VMEM((_CHUNK // 128, 128), jnp.float32),
            pltpu.VMEM((_CHUNK // 128, 128), jnp.float32),
            pltpu.VMEM((_L,), jnp.float32),
            pltpu.SemaphoreType.DMA,
        ],
    )(_sc_gather_body)


def _sc_gather_body(blocks_hbm, targets_hbm, tbl_hbm, lse_hbm, out_hbm,
                    b_v, t_v, idx_v, val_v, lseval_v, acc_v, sem):
    wid = lax.axis_index("s") * _NC + lax.axis_index("c")
    base = wid * _CHUNK
    n_rows = _CHUNK // 128
    for j in range(n_rows):
        pltpu.sync_copy(blocks_hbm.at[pl.ds(base + j * 128, 128)], b_v.at[j])
        pltpu.sync_copy(targets_hbm.at[pl.ds(base + j * 128, 128)], t_v.at[j])

    for j in range(n_rows):
        for i in range(128 // _L):
            bb = b_v[j, pl.ds(i * _L, _L)]
            tt = t_v[j, pl.ds(i * _L, _L)]
            idx_v[j, pl.ds(i * _L, _L)] = bb * _V + tt
    copies = []
    for j in range(n_rows):
        copies.append(pltpu.async_copy(tbl_hbm.at[idx_v.at[j]], val_v.at[j], sem))
        copies.append(pltpu.async_copy(lse_hbm.at[b_v.at[j]], lseval_v.at[j], sem))
    for c in copies:
        c.wait()

    acc = jnp.zeros((_L,), jnp.float32)
    for j in range(n_rows):
        for i in range(128 // _L):
            sl = pl.ds(i * _L, _L)
            acc = acc + (lseval_v[j, sl] - val_v[j, sl])
    acc_v[...] = acc
    pltpu.sync_copy(acc_v, out_hbm.at[wid])


def kernel(blocks, targets, table):
    blocks_f = blocks.reshape(-1).astype(jnp.int32)
    targets_f = targets.reshape(-1).astype(jnp.int32)
    lse, flat = _row_lse(table)
    parts = _make_sc_gather()(blocks_f, targets_f, flat, lse)
    return jnp.sum(parts) / jnp.float32(_N)


# speedup vs baseline: 3.1245x; 1.1113x over previous
import functools

import jax
import jax.numpy as jnp
from jax import lax
from jax.experimental import pallas as pl
from jax.experimental.pallas import tpu as pltpu
from jax.experimental.pallas import tpu_sc as plsc

_V = 8192
_N = 8192
_ROWS_BLK = 128
_NC = 2
_NS = 16
_NW = _NC * _NS
_CHUNK = _N // _NW
_L = 16


def _lse_body(tbl_ref, out_ref, flat_ref):
    x = tbl_ref[...]
    m = jnp.max(x, axis=1)
    s = jnp.sum(jnp.exp(x - m[:, None]), axis=1)
    i = pl.program_id(0)
    out_ref[pl.ds(i, 1), :] = (m + jnp.log(s)).reshape(1, _ROWS_BLK)
    xb = jax.lax.bitcast_convert_type(x, jnp.int32)
    a = xb[: _ROWS_BLK // 2, :] + jnp.int32(0x8000)
    b = xb[_ROWS_BLK // 2 :, :] + jnp.int32(0x8000)
    w = jax.lax.shift_right_logical(a, 16) | (b & jnp.int32(-65536))
    flat_ref[...] = w.reshape(-1)


def _row_lse(table):
    grid = _V // _ROWS_BLK
    out, flat = pl.pallas_call(
        _lse_body,
        grid=(grid,),
        in_specs=[pl.BlockSpec((_ROWS_BLK, _V), lambda i: (i, 0))],
        out_specs=[
            pl.BlockSpec((grid, _ROWS_BLK), lambda i: (0, 0)),
            pl.BlockSpec((_ROWS_BLK * _V // 2,), lambda i: (i,)),
        ],
        out_shape=[
            jax.ShapeDtypeStruct((grid, _ROWS_BLK), jnp.float32),
            jax.ShapeDtypeStruct((_V * _V // 2,), jnp.int32),
        ],
    )(table)
    return out.reshape(-1), flat


@functools.cache
def _make_sc_gather():
    mesh = plsc.VectorSubcoreMesh(core_axis_name="c", subcore_axis_name="s")
    return functools.partial(
        pl.kernel,
        mesh=mesh,
        out_type=jax.ShapeDtypeStruct((_NW, _L), jnp.float32),
        scratch_types=[
            pltpu.VMEM((_CHUNK // 128, 128), jnp.int32),
            pltpu.VMEM((_CHUNK // 128, 128), jnp.int32),
            pltpu.VMEM((_CHUNK // 128, 128), jnp.int32),
            pltpu.VMEM((_CHUNK // 128, 128), jnp.int32),
            pltpu.VMEM((_CHUNK // 128, 128), jnp.float32),
            pltpu.VMEM((_L,), jnp.float32),
            pltpu.SemaphoreType.DMA,
        ],
    )(_sc_gather_body)


def _sc_gather_body(blocks_hbm, targets_hbm, tbl_hbm, lse_hbm, out_hbm,
                    b_v, t_v, idx_v, val_v, lseval_v, acc_v, sem):
    wid = lax.axis_index("s") * _NC + lax.axis_index("c")
    base = wid * _CHUNK
    n_rows = _CHUNK // 128
    for j in range(n_rows):
        pltpu.sync_copy(blocks_hbm.at[pl.ds(base + j * 128, 128)], b_v.at[j])
        pltpu.sync_copy(targets_hbm.at[pl.ds(base + j * 128, 128)], t_v.at[j])

    for j in range(n_rows):
        for i in range(128 // _L):
            bb = b_v[j, pl.ds(i * _L, _L)]
            tt = t_v[j, pl.ds(i * _L, _L)]
            wrow = ((bb >> 7) << 6) | (bb & 63)
            idx_v[j, pl.ds(i * _L, _L)] = wrow * _V + tt
    copies = []
    for j in range(n_rows):
        copies.append(pltpu.async_copy(tbl_hbm.at[idx_v.at[j]], val_v.at[j], sem))
        copies.append(pltpu.async_copy(lse_hbm.at[b_v.at[j]], lseval_v.at[j], sem))
    for c in copies:
        c.wait()

    acc = jnp.zeros((_L,), jnp.float32)
    for j in range(n_rows):
        for i in range(128 // _L):
            sl = pl.ds(i * _L, _L)
            w = val_v[j, sl]
            odd = (b_v[j, sl] & 64) == 64
            bits = jnp.where(odd, w & jnp.int32(-65536), w << 16)
            tgt = jax.lax.bitcast_convert_type(bits, jnp.float32)
            acc = acc + (lseval_v[j, sl] - tgt)
    acc_v[...] = acc
    pltpu.sync_copy(acc_v, out_hbm.at[wid])


def kernel(blocks, targets, table):
    blocks_f = blocks.reshape(-1).astype(jnp.int32)
    targets_f = targets.reshape(-1).astype(jnp.int32)
    lse, flat = _row_lse(table)
    parts = _make_sc_gather()(blocks_f, targets_f, flat, lse)
    return jnp.sum(parts) / jnp.float32(_N)


# speedup vs baseline: 3.4492x; 1.1039x over previous
import functools

import jax
import jax.numpy as jnp
from jax import lax
from jax.experimental import pallas as pl
from jax.experimental.pallas import tpu as pltpu
from jax.experimental.pallas import tpu_sc as plsc

_V = 8192
_N = 8192
_ROWS_BLK = 256
_NC = 2
_NS = 16
_NW = _NC * _NS
_CHUNK = _N // _NW
_L = 16


def _lse_body(tbl_ref, out_ref, flat_ref):
    x = tbl_ref[...]
    m = jnp.max(x, axis=1)
    s = jnp.sum(jnp.exp(x - m[:, None]), axis=1)
    i = pl.program_id(0)
    out_ref[pl.ds(i, 1), :] = (m + jnp.log(s)).reshape(1, _ROWS_BLK)
    xb = jax.lax.bitcast_convert_type(x, jnp.int32)
    a = xb[: _ROWS_BLK // 2, :] + jnp.int32(0x8000)
    b = xb[_ROWS_BLK // 2 :, :] + jnp.int32(0x8000)
    w = jax.lax.shift_right_logical(a, 16) | (b & jnp.int32(-65536))
    flat_ref[...] = w.reshape(-1)


def _row_lse(table):
    grid = _V // _ROWS_BLK
    out, flat = pl.pallas_call(
        _lse_body,
        grid=(grid,),
        in_specs=[pl.BlockSpec((_ROWS_BLK, _V), lambda i: (i, 0))],
        out_specs=[
            pl.BlockSpec((grid, _ROWS_BLK), lambda i: (0, 0)),
            pl.BlockSpec((_ROWS_BLK * _V // 2,), lambda i: (i,)),
        ],
        out_shape=[
            jax.ShapeDtypeStruct((grid, _ROWS_BLK), jnp.float32),
            jax.ShapeDtypeStruct((_V * _V // 2,), jnp.int32),
        ],
    )(table)
    return out.reshape(-1), flat


@functools.cache
def _make_sc_gather():
    mesh = plsc.VectorSubcoreMesh(core_axis_name="c", subcore_axis_name="s")
    return functools.partial(
        pl.kernel,
        mesh=mesh,
        out_type=jax.ShapeDtypeStruct((_NW, _L), jnp.float32),
        scratch_types=[
            pltpu.VMEM((_CHUNK // 128, 128), jnp.int32),
            pltpu.VMEM((_CHUNK // 128, 128), jnp.int32),
            pltpu.VMEM((_CHUNK // 128, 128), jnp.int32),
            pltpu.VMEM((_CHUNK // 128, 128), jnp.int32),
            pltpu.VMEM((_CHUNK // 128, 128), jnp.float32),
            pltpu.VMEM((_L,), jnp.float32),
            pltpu.SemaphoreType.DMA,
        ],
    )(_sc_gather_body)


def _sc_gather_body(blocks_hbm, targets_hbm, tbl_hbm, lse_hbm, out_hbm,
                    b_v, t_v, idx_v, val_v, lseval_v, acc_v, sem):
    wid = lax.axis_index("s") * _NC + lax.axis_index("c")
    base = wid * _CHUNK
    n_rows = _CHUNK // 128
    for j in range(n_rows):
        pltpu.sync_copy(blocks_hbm.at[pl.ds(base + j * 128, 128)], b_v.at[j])
        pltpu.sync_copy(targets_hbm.at[pl.ds(base + j * 128, 128)], t_v.at[j])

    for j in range(n_rows):
        for i in range(128 // _L):
            bb = b_v[j, pl.ds(i * _L, _L)]
            tt = t_v[j, pl.ds(i * _L, _L)]
            wrow = ((bb >> 7) << 6) | (bb & 63)
            idx_v[j, pl.ds(i * _L, _L)] = wrow * _V + tt
    copies = []
    for j in range(n_rows):
        copies.append(pltpu.async_copy(tbl_hbm.at[idx_v.at[j]], val_v.at[j], sem))
        copies.append(pltpu.async_copy(lse_hbm.at[b_v.at[j]], lseval_v.at[j], sem))
    for c in copies:
        c.wait()

    acc = jnp.zeros((_L,), jnp.float32)
    for j in range(n_rows):
        for i in range(128 // _L):
            sl = pl.ds(i * _L, _L)
            w = val_v[j, sl]
            odd = (b_v[j, sl] & 64) == 64
            bits = jnp.where(odd, w & jnp.int32(-65536), w << 16)
            tgt = jax.lax.bitcast_convert_type(bits, jnp.float32)
            acc = acc + (lseval_v[j, sl] - tgt)
    acc_v[...] = acc
    pltpu.sync_copy(acc_v, out_hbm.at[wid])


def kernel(blocks, targets, table):
    blocks_f = blocks.reshape(-1).astype(jnp.int32)
    targets_f = targets.reshape(-1).astype(jnp.int32)
    lse, flat = _row_lse(table)
    parts = _make_sc_gather()(blocks_f, targets_f, flat, lse)
    return jnp.sum(parts) / jnp.float32(_N)
